# Initial kernel scaffold; baseline (speedup 1.0000x reference)
import jax
import jax.numpy as jnp
from jax.experimental import pallas as pl


def kernel(x, edge_index, Wl1, bl1, Wr1, Wl2, bl2, Wr2, Wl3, bl3, Wr3, Wh1, bh1, Wh2, bh2):
    raise NotImplementedError("write your pallas kernel here")


# speedup vs baseline: 10.0390x; 10.0390x over previous
import functools

import jax
import jax.numpy as jnp
from jax import lax
from jax.experimental import pallas as pl
from jax.experimental.pallas import tpu as pltpu
from jax.experimental.pallas import tpu_sc as plsc

N = 10000
E = 320000
D = 128
H = 32

NC = 2
NS = 16
NW = NC * NS
CHUNK = 80
NCH = 128
EPW = NCH * CHUNK
E_PAD = NW * EPW
NBUF = 4
NPAD = 10240
ZR = NPAD // NS
CW = 16
BR = 2048


def _sc_agg_counts_body(y, srcr, dstr, z32, z16, ones_h, out_sum, out_cnt,
                        sidx, didx, rows, acc, s0, s1, s2, s3, ones_v, cacc):
    cid = lax.axis_index("c")
    sid = lax.axis_index("s")
    wid = sid * NC + cid
    pltpu.sync_copy(z32, acc.at[pl.ds(sid * ZR, ZR)])
    pltpu.sync_copy(z16, cacc.at[pl.ds(sid * ZR, ZR)])
    pltpu.sync_copy(srcr.at[wid], sidx)
    pltpu.sync_copy(dstr.at[wid], didx)
    pltpu.sync_copy(ones_h, ones_v)
    plsc.subcore_barrier()
    sems = [s0, s1, s2, s3]
    for b in range(NBUF):
        pltpu.async_copy(y.at[sidx.at[b]], rows.at[b], sems[b])

    def group(g, carry):
        for b in range(NBUF):
            j = g * NBUF + b
            pltpu.make_async_copy(y.at[pl.ds(0, CHUNK)], rows.at[b], sems[b]).wait()
            pltpu.sync_copy(rows.at[b], acc.at[didx.at[j]], add=True)
            pltpu.sync_copy(ones_v, cacc.at[didx.at[j]], add=True)

            @pl.when(j + NBUF < NCH)
            def _():
                pltpu.async_copy(y.at[sidx.at[j + NBUF]], rows.at[b], sems[b])
        return carry

    lax.fori_loop(0, NCH // NBUF, group, 0)
    plsc.subcore_barrier()
    pltpu.sync_copy(acc.at[pl.ds(sid * ZR, ZR)], out_sum.at[cid, pl.ds(sid * ZR, ZR)])
    pltpu.sync_copy(cacc.at[pl.ds(sid * ZR, ZR)], out_cnt.at[cid, pl.ds(sid * ZR, ZR)])


def _sc_agg_body(y, srcr, dstr, z32, out_sum,
                 sidx, didx, rows, acc, s0, s1, s2, s3):
    cid = lax.axis_index("c")
    sid = lax.axis_index("s")
    wid = sid * NC + cid
    pltpu.sync_copy(z32, acc.at[pl.ds(sid * ZR, ZR)])
    pltpu.sync_copy(srcr.at[wid], sidx)
    pltpu.sync_copy(dstr.at[wid], didx)
    plsc.subcore_barrier()
    sems = [s0, s1, s2, s3]
    for b in range(NBUF):
        pltpu.async_copy(y.at[sidx.at[b]], rows.at[b], sems[b])

    def group(g, carry):
        for b in range(NBUF):
            j = g * NBUF + b
            pltpu.make_async_copy(y.at[pl.ds(0, CHUNK)], rows.at[b], sems[b]).wait()
            pltpu.sync_copy(rows.at[b], acc.at[didx.at[j]], add=True)

            @pl.when(j + NBUF < NCH)
            def _():
                pltpu.async_copy(y.at[sidx.at[j + NBUF]], rows.at[b], sems[b])
        return carry

    lax.fori_loop(0, NCH // NBUF, group, 0)
    plsc.subcore_barrier()
    pltpu.sync_copy(acc.at[pl.ds(sid * ZR, ZR)], out_sum.at[cid, pl.ds(sid * ZR, ZR)])


@functools.cache
def _sc_kernels():
    mesh = plsc.VectorSubcoreMesh(core_axis_name="c", subcore_axis_name="s",
                                  num_cores=NC, num_subcores=NS)
    common_scratch = [
        pltpu.VMEM((NCH, CHUNK), jnp.int32),
        pltpu.VMEM((NCH, CHUNK), jnp.int32),
        pltpu.VMEM((NBUF, CHUNK, H), jnp.float32),
        pltpu.VMEM_SHARED((NPAD, H), jnp.float32),
        pltpu.SemaphoreType.DMA,
        pltpu.SemaphoreType.DMA,
        pltpu.SemaphoreType.DMA,
        pltpu.SemaphoreType.DMA,
    ]
    agg_counts = pl.kernel(
        _sc_agg_counts_body,
        out_type=[jax.ShapeDtypeStruct((NC, NPAD, H), jnp.float32),
                  jax.ShapeDtypeStruct((NC, NPAD, CW), jnp.float32)],
        mesh=mesh,
        compiler_params=pltpu.CompilerParams(use_tc_tiling_on_sc=False),
        scratch_types=common_scratch + [
            pltpu.VMEM((CHUNK, CW), jnp.float32),
            pltpu.VMEM_SHARED((NPAD, CW), jnp.float32),
        ],
    )
    agg = pl.kernel(
        _sc_agg_body,
        out_type=[jax.ShapeDtypeStruct((NC, NPAD, H), jnp.float32)],
        mesh=mesh,
        compiler_params=pltpu.CompilerParams(use_tc_tiling_on_sc=False),
        scratch_types=common_scratch,
    )
    return agg_counts, agg


def _proj_body(x, wlT, wrT, bl, y_ref, z_ref):
    xb = x[...]
    y_ref[...] = jnp.dot(xb, wlT[...], preferred_element_type=jnp.float32)
    z_ref[...] = jnp.dot(xb, wrT[...], preferred_element_type=jnp.float32) + bl[...]


_proj = pl.pallas_call(
    _proj_body,
    grid=(NPAD // BR,),
    in_specs=[pl.BlockSpec((BR, D), lambda i: (i, 0)),
              pl.BlockSpec((D, H), lambda i: (0, 0)),
              pl.BlockSpec((D, H), lambda i: (0, 0)),
              pl.BlockSpec((1, H), lambda i: (0, 0))],
    out_specs=[pl.BlockSpec((BR, H), lambda i: (i, 0)),
               pl.BlockSpec((BR, H), lambda i: (i, 0))],
    out_shape=[jax.ShapeDtypeStruct((NPAD, H), jnp.float32)] * 2,
)


def _comb1_body(pa, pb, c0, c1, z, wlT, wrT, bl, y_ref, z_ref, inv_ref):
    cnt = c0[...] + c1[...]
    inv = 1.0 / jnp.maximum(cnt, 1.0)
    h = jnp.maximum((pa[...] + pb[...]) * inv[:, 0:1] + z[...], 0.0)
    y_ref[...] = jnp.dot(h, wlT[...], preferred_element_type=jnp.float32)
    z_ref[...] = jnp.dot(h, wrT[...], preferred_element_type=jnp.float32) + bl[...]
    inv_ref[...] = inv


_comb1 = pl.pallas_call(
    _comb1_body,
    grid=(NPAD // BR,),
    in_specs=[pl.BlockSpec((BR, H), lambda i: (i, 0)),
              pl.BlockSpec((BR, H), lambda i: (i, 0)),
              pl.BlockSpec((BR, CW), lambda i: (i, 0)),
              pl.BlockSpec((BR, CW), lambda i: (i, 0)),
              pl.BlockSpec((BR, H), lambda i: (i, 0)),
              pl.BlockSpec((H, H), lambda i: (0, 0)),
              pl.BlockSpec((H, H), lambda i: (0, 0)),
              pl.BlockSpec((1, H), lambda i: (0, 0))],
    out_specs=[pl.BlockSpec((BR, H), lambda i: (i, 0)),
               pl.BlockSpec((BR, H), lambda i: (i, 0)),
               pl.BlockSpec((BR, CW), lambda i: (i, 0))],
    out_shape=[jax.ShapeDtypeStruct((NPAD, H), jnp.float32),
               jax.ShapeDtypeStruct((NPAD, H), jnp.float32),
               jax.ShapeDtypeStruct((NPAD, CW), jnp.float32)],
)


def _comb2_body(pa, pb, inv_in, z, wlT, wrT, bl, y_ref, z_ref):
    inv = inv_in[...]
    h = jnp.maximum((pa[...] + pb[...]) * inv[:, 0:1] + z[...], 0.0)
    y_ref[...] = jnp.dot(h, wlT[...], preferred_element_type=jnp.float32)
    z_ref[...] = jnp.dot(h, wrT[...], preferred_element_type=jnp.float32) + bl[...]


_comb2 = pl.pallas_call(
    _comb2_body,
    grid=(NPAD // BR,),
    in_specs=[pl.BlockSpec((BR, H), lambda i: (i, 0)),
              pl.BlockSpec((BR, H), lambda i: (i, 0)),
              pl.BlockSpec((BR, CW), lambda i: (i, 0)),
              pl.BlockSpec((BR, H), lambda i: (i, 0)),
              pl.BlockSpec((H, H), lambda i: (0, 0)),
              pl.BlockSpec((H, H), lambda i: (0, 0)),
              pl.BlockSpec((1, H), lambda i: (0, 0))],
    out_specs=[pl.BlockSpec((BR, H), lambda i: (i, 0)),
               pl.BlockSpec((BR, H), lambda i: (i, 0))],
    out_shape=[jax.ShapeDtypeStruct((NPAD, H), jnp.float32)] * 2,
)


def _head_body(pa, pb, inv_in, z, wh1T, bh1, wh2T, bh2, o_ref):
    h = jnp.maximum((pa[...] + pb[...]) * inv_in[:, 0:1] + z[...], 0.0)
    t = jnp.maximum(jnp.dot(h, wh1T[...], preferred_element_type=jnp.float32) + bh1[...], 0.0)
    o_ref[...] = jnp.dot(t, wh2T[...], preferred_element_type=jnp.float32) + bh2[...]


_head = pl.pallas_call(
    _head_body,
    grid=(NPAD // BR,),
    in_specs=[pl.BlockSpec((BR, H), lambda i: (i, 0)),
              pl.BlockSpec((BR, H), lambda i: (i, 0)),
              pl.BlockSpec((BR, CW), lambda i: (i, 0)),
              pl.BlockSpec((BR, H), lambda i: (i, 0)),
              pl.BlockSpec((H, H), lambda i: (0, 0)),
              pl.BlockSpec((1, H), lambda i: (0, 0)),
              pl.BlockSpec((H, 8), lambda i: (0, 0)),
              pl.BlockSpec((1, 8), lambda i: (0, 0))],
    out_specs=pl.BlockSpec((BR, 8), lambda i: (i, 0)),
    out_shape=jax.ShapeDtypeStruct((NPAD, 8), jnp.float32),
)


def kernel(x, edge_index, Wl1, bl1, Wr1, Wl2, bl2, Wr2, Wl3, bl3, Wr3,
           Wh1, bh1, Wh2, bh2):
    src = edge_index[0].astype(jnp.int32)
    dst = edge_index[1].astype(jnp.int32)
    src_p = jnp.concatenate([src, jnp.zeros((E_PAD - E,), jnp.int32)]
                            ).reshape(NW, NCH, CHUNK)
    dst_p = jnp.concatenate([dst, jnp.full((E_PAD - E,), N, jnp.int32)]
                            ).reshape(NW, NCH, CHUNK)
    z32 = jnp.zeros((ZR, H), jnp.float32)
    z16 = jnp.zeros((ZR, CW), jnp.float32)
    ones_h = jnp.ones((CHUNK, CW), jnp.float32)

    _sc_agg_counts, _sc_agg = _sc_kernels()
    xp = jnp.concatenate([x, jnp.zeros((NPAD - N, D), jnp.float32)])
    y1, zb1 = _proj(xp, Wl1.T, Wr1.T, bl1.reshape(1, H))
    psum1, pcnt1 = _sc_agg_counts(y1, src_p, dst_p, z32, z16, ones_h)
    y2, zb2, inv = _comb1(psum1[0], psum1[1], pcnt1[0], pcnt1[1], zb1,
                          Wl2.T, Wr2.T, bl2.reshape(1, H))
    (psum2,) = _sc_agg(y2, src_p, dst_p, z32)
    y3, zb3 = _comb2(psum2[0], psum2[1], inv, zb2, Wl3.T, Wr3.T, bl3.reshape(1, H))
    (psum3,) = _sc_agg(y3, src_p, dst_p, z32)
    wh2T = jnp.zeros((H, 8), jnp.float32).at[:, :3].set(Wh2.T)
    bh2p = jnp.zeros((1, 8), jnp.float32).at[0, :3].set(bh2)
    out8 = _head(psum3[0], psum3[1], inv, zb3, Wh1.T, bh1.reshape(1, H),
                 wh2T, bh2p)
    return out8[:N, :3]


# speedup vs baseline: 20.2095x; 2.0131x over previous
import functools

import jax
import jax.numpy as jnp
from jax import lax
from jax.experimental import pallas as pl
from jax.experimental.pallas import tpu as pltpu
from jax.experimental.pallas import tpu_sc as plsc

N = 10000
E = 320000
D = 128
H = 32

NC = 2
NS = 16
NW = NC * NS
CHUNK = 80
NCH = 125
EPW = NCH * CHUNK
NBUF = 5
NPAD = 10240
ZR = NPAD // NS
SEG = N // NS
CW = 16
BR = 2000


def _sc_agg_counts_body(y, ei4, z32, z16, ones_h, out_sum, out_cnt,
                        sidx, didx, rows, ysp, acc, s0, s1, s2, s3, s4,
                        ones_v, cacc):
    cid = lax.axis_index("c")
    sid = lax.axis_index("s")
    wid = sid * NC + cid
    pltpu.sync_copy(z32, acc.at[pl.ds(sid * ZR, ZR)])
    pltpu.sync_copy(z16, cacc.at[pl.ds(sid * ZR, ZR)])
    pltpu.sync_copy(y.at[pl.ds(sid * SEG, SEG)], ysp.at[pl.ds(sid * SEG, SEG)])
    pltpu.sync_copy(ei4.at[0, wid], sidx)
    pltpu.sync_copy(ei4.at[1, wid], didx)
    pltpu.sync_copy(ones_h, ones_v)
    plsc.subcore_barrier()
    sems = [s0, s1, s2, s3, s4]
    for b in range(NBUF):
        pltpu.async_copy(ysp.at[sidx.at[b]], rows.at[b], sems[b])

    def group(g, carry):
        for b in range(NBUF):
            j = g * NBUF + b
            pltpu.make_async_copy(y.at[pl.ds(0, CHUNK)], rows.at[b], sems[b]).wait()
            pltpu.sync_copy(rows.at[b], acc.at[didx.at[j]], add=True)
            pltpu.sync_copy(ones_v, cacc.at[didx.at[j]], add=True)

            @pl.when(j + NBUF < NCH)
            def _():
                pltpu.async_copy(ysp.at[sidx.at[j + NBUF]], rows.at[b], sems[b])
        return carry

    lax.fori_loop(0, NCH // NBUF, group, 0)
    plsc.subcore_barrier()
    pltpu.sync_copy(acc.at[pl.ds(sid * ZR, ZR)], out_sum.at[cid, pl.ds(sid * ZR, ZR)])
    pltpu.sync_copy(cacc.at[pl.ds(sid * ZR, ZR)], out_cnt.at[cid, pl.ds(sid * ZR, ZR)])


def _sc_agg_body(y, ei4, z32, out_sum,
                 sidx, didx, rows, ysp, acc, s0, s1, s2, s3, s4):
    cid = lax.axis_index("c")
    sid = lax.axis_index("s")
    wid = sid * NC + cid
    pltpu.sync_copy(z32, acc.at[pl.ds(sid * ZR, ZR)])
    pltpu.sync_copy(y.at[pl.ds(sid * SEG, SEG)], ysp.at[pl.ds(sid * SEG, SEG)])
    pltpu.sync_copy(ei4.at[0, wid], sidx)
    pltpu.sync_copy(ei4.at[1, wid], didx)
    plsc.subcore_barrier()
    sems = [s0, s1, s2, s3, s4]
    for b in range(NBUF):
        pltpu.async_copy(ysp.at[sidx.at[b]], rows.at[b], sems[b])

    def group(g, carry):
        for b in range(NBUF):
            j = g * NBUF + b
            pltpu.make_async_copy(y.at[pl.ds(0, CHUNK)], rows.at[b], sems[b]).wait()
            pltpu.sync_copy(rows.at[b], acc.at[didx.at[j]], add=True)

            @pl.when(j + NBUF < NCH)
            def _():
                pltpu.async_copy(ysp.at[sidx.at[j + NBUF]], rows.at[b], sems[b])
        return carry

    lax.fori_loop(0, NCH // NBUF, group, 0)
    plsc.subcore_barrier()
    pltpu.sync_copy(acc.at[pl.ds(sid * ZR, ZR)], out_sum.at[cid, pl.ds(sid * ZR, ZR)])


@functools.cache
def _sc_kernels():
    mesh = plsc.VectorSubcoreMesh(core_axis_name="c", subcore_axis_name="s",
                                  num_cores=NC, num_subcores=NS)
    common_scratch = [
        pltpu.VMEM((NCH, CHUNK), jnp.int32),
        pltpu.VMEM((NCH, CHUNK), jnp.int32),
        pltpu.VMEM((NBUF, CHUNK, H), jnp.float32),
        pltpu.VMEM_SHARED((N, H), jnp.float32),
        pltpu.VMEM_SHARED((NPAD, H), jnp.float32),
        pltpu.SemaphoreType.DMA,
        pltpu.SemaphoreType.DMA,
        pltpu.SemaphoreType.DMA,
        pltpu.SemaphoreType.DMA,
        pltpu.SemaphoreType.DMA,
    ]
    agg_counts = pl.kernel(
        _sc_agg_counts_body,
        out_type=[jax.ShapeDtypeStruct((NC, NPAD, H), jnp.float32),
                  jax.ShapeDtypeStruct((NC, NPAD, CW), jnp.float32)],
        mesh=mesh,
        compiler_params=pltpu.CompilerParams(use_tc_tiling_on_sc=False),
        scratch_types=common_scratch + [
            pltpu.VMEM((CHUNK, CW), jnp.float32),
            pltpu.VMEM_SHARED((NPAD, CW), jnp.float32),
        ],
    )
    agg = pl.kernel(
        _sc_agg_body,
        out_type=[jax.ShapeDtypeStruct((NC, NPAD, H), jnp.float32)],
        mesh=mesh,
        compiler_params=pltpu.CompilerParams(use_tc_tiling_on_sc=False),
        scratch_types=common_scratch,
    )
    return agg_counts, agg


_ROW = lambda i: (i, 0)
_FULL = lambda i: (0, 0)
_P0 = lambda i: (0, i, 0)
_P1 = lambda i: (1, i, 0)


def _proj_body(x, wlT, wrT, bl, y_ref, z_ref):
    xb = x[...]
    y_ref[...] = jnp.dot(xb, wlT[...], preferred_element_type=jnp.float32)
    z_ref[...] = jnp.dot(xb, wrT[...], preferred_element_type=jnp.float32) + bl[...]


_proj = pl.pallas_call(
    _proj_body,
    grid=(N // BR,),
    in_specs=[pl.BlockSpec((BR, D), _ROW),
              pl.BlockSpec((D, H), _FULL),
              pl.BlockSpec((D, H), _FULL),
              pl.BlockSpec((1, H), _FULL)],
    out_specs=[pl.BlockSpec((BR, H), _ROW),
               pl.BlockSpec((BR, H), _ROW)],
    out_shape=[jax.ShapeDtypeStruct((N, H), jnp.float32)] * 2,
)


def _comb1_body(pa, pb, c0, c1, z, wlT, wrT, bl, y_ref, z_ref, inv_ref):
    cnt = c0[0] + c1[0]
    inv = 1.0 / jnp.maximum(cnt, 1.0)
    h = jnp.maximum((pa[0] + pb[0]) * inv[:, 0:1] + z[...], 0.0)
    y_ref[...] = jnp.dot(h, wlT[...], preferred_element_type=jnp.float32)
    z_ref[...] = jnp.dot(h, wrT[...], preferred_element_type=jnp.float32) + bl[...]
    inv_ref[...] = inv


_comb1 = pl.pallas_call(
    _comb1_body,
    grid=(N // BR,),
    in_specs=[pl.BlockSpec((1, BR, H), _P0),
              pl.BlockSpec((1, BR, H), _P1),
              pl.BlockSpec((1, BR, CW), _P0),
              pl.BlockSpec((1, BR, CW), _P1),
              pl.BlockSpec((BR, H), _ROW),
              pl.BlockSpec((H, H), _FULL),
              pl.BlockSpec((H, H), _FULL),
              pl.BlockSpec((1, H), _FULL)],
    out_specs=[pl.BlockSpec((BR, H), _ROW),
               pl.BlockSpec((BR, H), _ROW),
               pl.BlockSpec((BR, CW), _ROW)],
    out_shape=[jax.ShapeDtypeStruct((N, H), jnp.float32),
               jax.ShapeDtypeStruct((N, H), jnp.float32),
               jax.ShapeDtypeStruct((N, CW), jnp.float32)],
)


def _comb2_body(pa, pb, inv_in, z, wlT, wrT, bl, y_ref, z_ref):
    inv = inv_in[...]
    h = jnp.maximum((pa[0] + pb[0]) * inv[:, 0:1] + z[...], 0.0)
    y_ref[...] = jnp.dot(h, wlT[...], preferred_element_type=jnp.float32)
    z_ref[...] = jnp.dot(h, wrT[...], preferred_element_type=jnp.float32) + bl[...]


_comb2 = pl.pallas_call(
    _comb2_body,
    grid=(N // BR,),
    in_specs=[pl.BlockSpec((1, BR, H), _P0),
              pl.BlockSpec((1, BR, H), _P1),
              pl.BlockSpec((BR, CW), _ROW),
              pl.BlockSpec((BR, H), _ROW),
              pl.BlockSpec((H, H), _FULL),
              pl.BlockSpec((H, H), _FULL),
              pl.BlockSpec((1, H), _FULL)],
    out_specs=[pl.BlockSpec((BR, H), _ROW),
               pl.BlockSpec((BR, H), _ROW)],
    out_shape=[jax.ShapeDtypeStruct((N, H), jnp.float32)] * 2,
)


def _head_body(pa, pb, inv_in, z, wh1T, bh1, wh2T, bh2, o_ref):
    h = jnp.maximum((pa[0] + pb[0]) * inv_in[:, 0:1] + z[...], 0.0)
    t = jnp.maximum(jnp.dot(h, wh1T[...], preferred_element_type=jnp.float32) + bh1[...], 0.0)
    o_ref[...] = jnp.dot(t, wh2T[...], preferred_element_type=jnp.float32) + bh2[...]


_head = pl.pallas_call(
    _head_body,
    grid=(N // BR,),
    in_specs=[pl.BlockSpec((1, BR, H), _P0),
              pl.BlockSpec((1, BR, H), _P1),
              pl.BlockSpec((BR, CW), _ROW),
              pl.BlockSpec((BR, H), _ROW),
              pl.BlockSpec((H, H), _FULL),
              pl.BlockSpec((1, H), _FULL),
              pl.BlockSpec((H, 8), _FULL),
              pl.BlockSpec((1, 8), _FULL)],
    out_specs=pl.BlockSpec((BR, 8), _ROW),
    out_shape=jax.ShapeDtypeStruct((N, 8), jnp.float32),
)


def kernel(x, edge_index, Wl1, bl1, Wr1, Wl2, bl2, Wr2, Wl3, bl3, Wr3,
           Wh1, bh1, Wh2, bh2):
    _sc_agg_counts, _sc_agg = _sc_kernels()
    ei4 = edge_index.astype(jnp.int32).reshape(2, NW, NCH, CHUNK)
    z32 = jnp.zeros((ZR, H), jnp.float32)
    z16 = jnp.zeros((ZR, CW), jnp.float32)
    ones_h = jnp.ones((CHUNK, CW), jnp.float32)

    y1, zb1 = _proj(x, Wl1.T, Wr1.T, bl1.reshape(1, H))
    psum1, pcnt1 = _sc_agg_counts(y1, ei4, z32, z16, ones_h)
    y2, zb2, inv = _comb1(psum1, psum1, pcnt1, pcnt1, zb1,
                          Wl2.T, Wr2.T, bl2.reshape(1, H))
    (psum2,) = _sc_agg(y2, ei4, z32)
    y3, zb3 = _comb2(psum2, psum2, inv, zb2, Wl3.T, Wr3.T, bl3.reshape(1, H))
    (psum3,) = _sc_agg(y3, ei4, z32)
    wh2T = jnp.zeros((H, 8), jnp.float32).at[:, :3].set(Wh2.T)
    bh2p = jnp.zeros((1, 8), jnp.float32).at[0, :3].set(bh2)
    out8 = _head(psum3, psum3, inv, zb3, Wh1.T, bh1.reshape(1, H),
                 wh2T, bh2p)
    return out8[:, :3]


# speedup vs baseline: 20.4200x; 1.0104x over previous
import functools

import numpy as np

import jax
import jax.numpy as jnp
from jax import lax
from jax.experimental import pallas as pl
from jax.experimental.pallas import tpu as pltpu
from jax.experimental.pallas import tpu_sc as plsc

N = 10000
E = 320000
D = 128
H = 32

NC = 2
NS = 16
NW = NC * NS
CHUNK = 80
NCH = 125
EPW = NCH * CHUNK
NBUF = 5
NPAD = 10240
ZR = NPAD // NS
SEG = N // NS
CW = 8
BR = 2000


def _sc_agg_counts_body(y, ei4, z32, z16, ones_h, out_sum, out_cnt,
                        sidx, didx, rows, ysp, acc, s0, s1, s2, s3, s4,
                        ones_v, cacc):
    cid = lax.axis_index("c")
    sid = lax.axis_index("s")
    wid = sid * NC + cid
    pltpu.sync_copy(z32, acc.at[pl.ds(sid * ZR, ZR)])
    pltpu.sync_copy(z16, cacc.at[pl.ds(sid * ZR, ZR)])
    pltpu.sync_copy(y.at[pl.ds(sid * SEG, SEG)], ysp.at[pl.ds(sid * SEG, SEG)])
    pltpu.sync_copy(ei4.at[0, wid], sidx)
    pltpu.sync_copy(ei4.at[1, wid], didx)
    pltpu.sync_copy(ones_h, ones_v)
    plsc.subcore_barrier()
    sems = [s0, s1, s2, s3, s4]
    for b in range(NBUF):
        pltpu.async_copy(ysp.at[sidx.at[b]], rows.at[b], sems[b])

    def group(g, carry):
        for b in range(NBUF):
            j = g * NBUF + b
            pltpu.make_async_copy(y.at[pl.ds(0, CHUNK)], rows.at[b], sems[b]).wait()
            pltpu.sync_copy(rows.at[b], acc.at[didx.at[j]], add=True)
            pltpu.sync_copy(ones_v, cacc.at[didx.at[j]], add=True)

            @pl.when(j + NBUF < NCH)
            def _():
                pltpu.async_copy(ysp.at[sidx.at[j + NBUF]], rows.at[b], sems[b])
        return carry

    lax.fori_loop(0, NCH // NBUF, group, 0)
    plsc.subcore_barrier()
    pltpu.sync_copy(acc.at[pl.ds(sid * ZR, ZR)], out_sum.at[cid, pl.ds(sid * ZR, ZR)])
    pltpu.sync_copy(cacc.at[pl.ds(sid * ZR, ZR)], out_cnt.at[cid, pl.ds(sid * ZR, ZR)])


def _sc_agg_body(y, ei4, z32, out_sum,
                 sidx, didx, rows, ysp, acc, s0, s1, s2, s3, s4):
    cid = lax.axis_index("c")
    sid = lax.axis_index("s")
    wid = sid * NC + cid
    pltpu.sync_copy(z32, acc.at[pl.ds(sid * ZR, ZR)])
    pltpu.sync_copy(y.at[pl.ds(sid * SEG, SEG)], ysp.at[pl.ds(sid * SEG, SEG)])
    pltpu.sync_copy(ei4.at[0, wid], sidx)
    pltpu.sync_copy(ei4.at[1, wid], didx)
    plsc.subcore_barrier()
    sems = [s0, s1, s2, s3, s4]
    for b in range(NBUF):
        pltpu.async_copy(ysp.at[sidx.at[b]], rows.at[b], sems[b])

    def group(g, carry):
        for b in range(NBUF):
            j = g * NBUF + b
            pltpu.make_async_copy(y.at[pl.ds(0, CHUNK)], rows.at[b], sems[b]).wait()
            pltpu.sync_copy(rows.at[b], acc.at[didx.at[j]], add=True)

            @pl.when(j + NBUF < NCH)
            def _():
                pltpu.async_copy(ysp.at[sidx.at[j + NBUF]], rows.at[b], sems[b])
        return carry

    lax.fori_loop(0, NCH // NBUF, group, 0)
    plsc.subcore_barrier()
    pltpu.sync_copy(acc.at[pl.ds(sid * ZR, ZR)], out_sum.at[cid, pl.ds(sid * ZR, ZR)])


@functools.cache
def _sc_kernels():
    mesh = plsc.VectorSubcoreMesh(core_axis_name="c", subcore_axis_name="s",
                                  num_cores=NC, num_subcores=NS)
    common_scratch = [
        pltpu.VMEM((NCH, CHUNK), jnp.int32),
        pltpu.VMEM((NCH, CHUNK), jnp.int32),
        pltpu.VMEM((NBUF, CHUNK, H), jnp.float32),
        pltpu.VMEM_SHARED((N, H), jnp.float32),
        pltpu.VMEM_SHARED((NPAD, H), jnp.float32),
        pltpu.SemaphoreType.DMA,
        pltpu.SemaphoreType.DMA,
        pltpu.SemaphoreType.DMA,
        pltpu.SemaphoreType.DMA,
        pltpu.SemaphoreType.DMA,
    ]
    agg_counts = pl.kernel(
        _sc_agg_counts_body,
        out_type=[jax.ShapeDtypeStruct((NC, NPAD, H), jnp.float32),
                  jax.ShapeDtypeStruct((NC, NPAD, CW), jnp.float32)],
        mesh=mesh,
        compiler_params=pltpu.CompilerParams(use_tc_tiling_on_sc=False),
        scratch_types=common_scratch + [
            pltpu.VMEM((CHUNK, CW), jnp.float32),
            pltpu.VMEM_SHARED((NPAD, CW), jnp.float32),
        ],
    )
    agg = pl.kernel(
        _sc_agg_body,
        out_type=[jax.ShapeDtypeStruct((NC, NPAD, H), jnp.float32)],
        mesh=mesh,
        compiler_params=pltpu.CompilerParams(use_tc_tiling_on_sc=False),
        scratch_types=common_scratch,
    )
    return agg_counts, agg


_ROW = lambda i: (i, 0)
_FULL = lambda i: (0, 0)
_P0 = lambda i: (0, i, 0)
_P1 = lambda i: (1, i, 0)


def _proj_body(x, wlT, wrT, bl, y_ref, z_ref):
    xb = x[...]
    y_ref[...] = jnp.dot(xb, wlT[...], preferred_element_type=jnp.float32)
    z_ref[...] = jnp.dot(xb, wrT[...], preferred_element_type=jnp.float32) + bl[...]


_proj = pl.pallas_call(
    _proj_body,
    grid=(N // BR,),
    in_specs=[pl.BlockSpec((BR, D), _ROW),
              pl.BlockSpec((D, H), _FULL),
              pl.BlockSpec((D, H), _FULL),
              pl.BlockSpec((1, H), _FULL)],
    out_specs=[pl.BlockSpec((BR, H), _ROW),
               pl.BlockSpec((BR, H), _ROW)],
    out_shape=[jax.ShapeDtypeStruct((N, H), jnp.float32)] * 2,
)


def _comb1_body(pa, pb, c0, c1, z, wlT, wrT, bl, y_ref, z_ref, inv_ref):
    cnt = c0[0] + c1[0]
    inv = 1.0 / jnp.maximum(cnt, 1.0)
    h = jnp.maximum((pa[0] + pb[0]) * inv[:, 0:1] + z[...], 0.0)
    y_ref[...] = jnp.dot(h, wlT[...], preferred_element_type=jnp.float32)
    z_ref[...] = jnp.dot(h, wrT[...], preferred_element_type=jnp.float32) + bl[...]
    inv_ref[...] = inv


_comb1 = pl.pallas_call(
    _comb1_body,
    grid=(N // BR,),
    in_specs=[pl.BlockSpec((1, BR, H), _P0),
              pl.BlockSpec((1, BR, H), _P1),
              pl.BlockSpec((1, BR, CW), _P0),
              pl.BlockSpec((1, BR, CW), _P1),
              pl.BlockSpec((BR, H), _ROW),
              pl.BlockSpec((H, H), _FULL),
              pl.BlockSpec((H, H), _FULL),
              pl.BlockSpec((1, H), _FULL)],
    out_specs=[pl.BlockSpec((BR, H), _ROW),
               pl.BlockSpec((BR, H), _ROW),
               pl.BlockSpec((BR, CW), _ROW)],
    out_shape=[jax.ShapeDtypeStruct((N, H), jnp.float32),
               jax.ShapeDtypeStruct((N, H), jnp.float32),
               jax.ShapeDtypeStruct((N, CW), jnp.float32)],
)


def _comb2_body(pa, pb, inv_in, z, wlT, wrT, bl, y_ref, z_ref):
    inv = inv_in[...]
    h = jnp.maximum((pa[0] + pb[0]) * inv[:, 0:1] + z[...], 0.0)
    y_ref[...] = jnp.dot(h, wlT[...], preferred_element_type=jnp.float32)
    z_ref[...] = jnp.dot(h, wrT[...], preferred_element_type=jnp.float32) + bl[...]


_comb2 = pl.pallas_call(
    _comb2_body,
    grid=(N // BR,),
    in_specs=[pl.BlockSpec((1, BR, H), _P0),
              pl.BlockSpec((1, BR, H), _P1),
              pl.BlockSpec((BR, CW), _ROW),
              pl.BlockSpec((BR, H), _ROW),
              pl.BlockSpec((H, H), _FULL),
              pl.BlockSpec((H, H), _FULL),
              pl.BlockSpec((1, H), _FULL)],
    out_specs=[pl.BlockSpec((BR, H), _ROW),
               pl.BlockSpec((BR, H), _ROW)],
    out_shape=[jax.ShapeDtypeStruct((N, H), jnp.float32)] * 2,
)


def _head_body(pa, pb, inv_in, z, wh1T, bh1, wh2T, bh2, o_ref):
    h = jnp.maximum((pa[0] + pb[0]) * inv_in[:, 0:1] + z[...], 0.0)
    t = jnp.maximum(jnp.dot(h, wh1T[...], preferred_element_type=jnp.float32) + bh1[...], 0.0)
    o_ref[...] = jnp.dot(t, wh2T[...], preferred_element_type=jnp.float32) + bh2[...]


_head = pl.pallas_call(
    _head_body,
    grid=(N // BR,),
    in_specs=[pl.BlockSpec((1, BR, H), _P0),
              pl.BlockSpec((1, BR, H), _P1),
              pl.BlockSpec((BR, CW), _ROW),
              pl.BlockSpec((BR, H), _ROW),
              pl.BlockSpec((H, H), _FULL),
              pl.BlockSpec((1, H), _FULL),
              pl.BlockSpec((H, 8), _FULL),
              pl.BlockSpec((1, 8), _FULL)],
    out_specs=pl.BlockSpec((BR, 8), _ROW),
    out_shape=jax.ShapeDtypeStruct((N, 8), jnp.float32),
)


def kernel(x, edge_index, Wl1, bl1, Wr1, Wl2, bl2, Wr2, Wl3, bl3, Wr3,
           Wh1, bh1, Wh2, bh2):
    _sc_agg_counts, _sc_agg = _sc_kernels()
    ei4 = edge_index.astype(jnp.int32).reshape(2, NW, NCH, CHUNK)
    z32 = np.zeros((ZR, H), np.float32)
    z16 = np.zeros((ZR, CW), np.float32)
    ones_h = np.ones((CHUNK, CW), np.float32)

    y1, zb1 = _proj(x, Wl1.T, Wr1.T, bl1.reshape(1, H))
    psum1, pcnt1 = _sc_agg_counts(y1, ei4, z32, z16, ones_h)
    y2, zb2, inv = _comb1(psum1, psum1, pcnt1, pcnt1, zb1,
                          Wl2.T, Wr2.T, bl2.reshape(1, H))
    (psum2,) = _sc_agg(y2, ei4, z32)
    y3, zb3 = _comb2(psum2, psum2, inv, zb2, Wl3.T, Wr3.T, bl3.reshape(1, H))
    (psum3,) = _sc_agg(y3, ei4, z32)
    wh2T = jnp.zeros((H, 8), jnp.float32).at[:, :3].set(Wh2.T)
    bh2p = jnp.zeros((1, 8), jnp.float32).at[0, :3].set(bh2)
    out8 = _head(psum3, psum3, inv, zb3, Wh1.T, bh1.reshape(1, H),
                 wh2T, bh2p)
    return out8[:, :3]


# speedup vs baseline: 22.6726x; 1.1103x over previous
import functools

import numpy as np

import jax
import jax.numpy as jnp
from jax import lax
from jax.experimental import pallas as pl
from jax.experimental.pallas import tpu as pltpu
from jax.experimental.pallas import tpu_sc as plsc

N = 10000
E = 320000
D = 128
H = 32
P = 4
N4 = N // P
LW = P * H

NC = 2
NS = 16
NW = NC * NS
CHUNK = 128
NCH = 80
EPW = NCH * CHUNK
EPWR = E // NW
NBUF = 5
NPAD = 10240
NP4 = NPAD // P
ZR = NPAD // NS
SEG = N // NS
CW = 32
BR = 2000
B4 = BR // P


def _sc_agg_counts_body(y, ei4, z32, ones_h, out_sum, out_cnt,
                        sidx, didx, rows, ysp, acc, s0, s1, s2, s3, s4,
                        ones_v, cacc):
    cid = lax.axis_index("c")
    sid = lax.axis_index("s")
    wid = sid * NC + cid
    pltpu.sync_copy(z32, acc.at[pl.ds(sid * ZR, ZR)])
    pltpu.sync_copy(z32, cacc.at[pl.ds(sid * ZR, ZR)])
    pltpu.sync_copy(y.at[pl.ds(sid * SEG, SEG)], ysp.at[pl.ds(sid * SEG, SEG)])
    pltpu.sync_copy(ei4.at[0, wid], sidx)
    pltpu.sync_copy(ei4.at[1, wid], didx)
    pltpu.sync_copy(ones_h, ones_v)
    plsc.subcore_barrier()
    sems = [s0, s1, s2, s3, s4]
    for b in range(NBUF):
        pltpu.async_copy(ysp.at[sidx.at[b]], rows.at[b], sems[b])

    def group(g, carry):
        for b in range(NBUF):
            j = g * NBUF + b
            pltpu.make_async_copy(y.at[pl.ds(0, CHUNK)], rows.at[b], sems[b]).wait()
            pltpu.sync_copy(rows.at[b], acc.at[didx.at[j]], add=True)
            pltpu.sync_copy(ones_v, cacc.at[didx.at[j]], add=True)

            @pl.when(j + NBUF < NCH)
            def _():
                pltpu.async_copy(ysp.at[sidx.at[j + NBUF]], rows.at[b], sems[b])
        return carry

    lax.fori_loop(0, NCH // NBUF, group, 0)
    plsc.subcore_barrier()
    pltpu.sync_copy(acc.at[pl.ds(sid * ZR, ZR)], out_sum.at[cid, pl.ds(sid * ZR, ZR)])
    pltpu.sync_copy(cacc.at[pl.ds(sid * ZR, ZR)], out_cnt.at[cid, pl.ds(sid * ZR, ZR)])


def _sc_agg_body(y, ei4, z32, out_sum,
                 sidx, didx, rows, ysp, acc, s0, s1, s2, s3, s4):
    cid = lax.axis_index("c")
    sid = lax.axis_index("s")
    wid = sid * NC + cid
    pltpu.sync_copy(z32, acc.at[pl.ds(sid * ZR, ZR)])
    pltpu.sync_copy(y.at[pl.ds(sid * SEG, SEG)], ysp.at[pl.ds(sid * SEG, SEG)])
    pltpu.sync_copy(ei4.at[0, wid], sidx)
    pltpu.sync_copy(ei4.at[1, wid], didx)
    plsc.subcore_barrier()
    sems = [s0, s1, s2, s3, s4]
    for b in range(NBUF):
        pltpu.async_copy(ysp.at[sidx.at[b]], rows.at[b], sems[b])

    def group(g, carry):
        for b in range(NBUF):
            j = g * NBUF + b
            pltpu.make_async_copy(y.at[pl.ds(0, CHUNK)], rows.at[b], sems[b]).wait()
            pltpu.sync_copy(rows.at[b], acc.at[didx.at[j]], add=True)

            @pl.when(j + NBUF < NCH)
            def _():
                pltpu.async_copy(ysp.at[sidx.at[j + NBUF]], rows.at[b], sems[b])
        return carry

    lax.fori_loop(0, NCH // NBUF, group, 0)
    plsc.subcore_barrier()
    pltpu.sync_copy(acc.at[pl.ds(sid * ZR, ZR)], out_sum.at[cid, pl.ds(sid * ZR, ZR)])


@functools.cache
def _sc_kernels():
    mesh = plsc.VectorSubcoreMesh(core_axis_name="c", subcore_axis_name="s",
                                  num_cores=NC, num_subcores=NS)
    common_scratch = [
        pltpu.VMEM((NCH, CHUNK), jnp.int32),
        pltpu.VMEM((NCH, CHUNK), jnp.int32),
        pltpu.VMEM((NBUF, CHUNK, H), jnp.float32),
        pltpu.VMEM_SHARED((N, H), jnp.float32),
        pltpu.VMEM_SHARED((NPAD, H), jnp.float32),
        pltpu.SemaphoreType.DMA,
        pltpu.SemaphoreType.DMA,
        pltpu.SemaphoreType.DMA,
        pltpu.SemaphoreType.DMA,
        pltpu.SemaphoreType.DMA,
    ]
    agg_counts = pl.kernel(
        _sc_agg_counts_body,
        out_type=[jax.ShapeDtypeStruct((NC, NPAD, H), jnp.float32),
                  jax.ShapeDtypeStruct((NC, NPAD, CW), jnp.float32)],
        mesh=mesh,
        compiler_params=pltpu.CompilerParams(use_tc_tiling_on_sc=False),
        scratch_types=common_scratch + [
            pltpu.VMEM((CHUNK, CW), jnp.float32),
            pltpu.VMEM_SHARED((NPAD, CW), jnp.float32),
        ],
    )
    agg = pl.kernel(
        _sc_agg_body,
        out_type=[jax.ShapeDtypeStruct((NC, NPAD, H), jnp.float32)],
        mesh=mesh,
        compiler_params=pltpu.CompilerParams(use_tc_tiling_on_sc=False),
        scratch_types=common_scratch,
    )
    return agg_counts, agg


def _proj_body(x, wlT, wrT, bl, y_ref, z_ref):
    xb = x[...]
    y_ref[...] = jnp.dot(xb, wlT[...], preferred_element_type=jnp.float32)
    z_ref[...] = jnp.dot(xb, wrT[...], preferred_element_type=jnp.float32) + bl[...]


_proj = pl.pallas_call(
    _proj_body,
    out_shape=[jax.ShapeDtypeStruct((N, H), jnp.float32)] * 2,
)


def _comb_body(ps, pc, z, wlbd, wrbd, blp, y_ref, z_ref):
    inv = 1.0 / jnp.maximum(pc[0] + pc[1], 1.0)
    h = jnp.maximum((ps[0] + ps[1]) * inv + z[...], 0.0)
    y_ref[...] = jnp.dot(h, wlbd[...], preferred_element_type=jnp.float32)
    z_ref[...] = jnp.dot(h, wrbd[...], preferred_element_type=jnp.float32) + blp[...]


_comb = pl.pallas_call(
    _comb_body,
    out_shape=[jax.ShapeDtypeStruct((N4, LW), jnp.float32)] * 2,
)


def _head_body(ps, pc, z, wh1bd, bh1p, wh2bd, bh2p, o_ref):
    inv = 1.0 / jnp.maximum(pc[0] + pc[1], 1.0)
    h = jnp.maximum((ps[0] + ps[1]) * inv + z[...], 0.0)
    t = jnp.maximum(jnp.dot(h, wh1bd[...], preferred_element_type=jnp.float32)
                    + bh1p[...], 0.0)
    o_ref[...] = jnp.dot(t, wh2bd[...], preferred_element_type=jnp.float32) + bh2p[...]


_head = pl.pallas_call(
    _head_body,
    out_shape=jax.ShapeDtypeStruct((N4, LW), jnp.float32),
)


_EPAD = np.stack([
    np.zeros((NW, EPW - EPWR), np.int32),
    np.broadcast_to(N + np.arange(EPW - EPWR, dtype=np.int32), (NW, EPW - EPWR)),
])


def _bd4(w):
    return jax.scipy.linalg.block_diag(w, w, w, w)


def _tile4(b):
    return jnp.tile(b, P).reshape(1, LW)


def kernel(x, edge_index, Wl1, bl1, Wr1, Wl2, bl2, Wr2, Wl3, bl3, Wr3,
           Wh1, bh1, Wh2, bh2):
    _sc_agg_counts, _sc_agg = _sc_kernels()
    ei = edge_index.astype(jnp.int32).reshape(2, NW, EPWR)
    ei4 = jnp.concatenate([ei, _EPAD], axis=2).reshape(2, NW, NCH, CHUNK)
    z32 = np.zeros((ZR, H), np.float32)
    ones_h = np.ones((CHUNK, CW), np.float32)

    y1, zb1 = _proj(x, Wl1.T, Wr1.T, bl1.reshape(1, H))
    psum1, pcnt1 = _sc_agg_counts(y1, ei4, z32, ones_h)
    ps1 = psum1.reshape(NC, NP4, LW)[:, :N4]
    pc1 = pcnt1.reshape(NC, NP4, LW)[:, :N4]
    y2p, zb2 = _comb(ps1, pc1, zb1.reshape(N4, LW),
                     _bd4(Wl2.T), _bd4(Wr2.T), _tile4(bl2))
    (psum2,) = _sc_agg(y2p.reshape(N, H), ei4, z32)
    ps2 = psum2.reshape(NC, NP4, LW)[:, :N4]
    y3p, zb3 = _comb(ps2, pc1, zb2, _bd4(Wl3.T), _bd4(Wr3.T), _tile4(bl3))
    (psum3,) = _sc_agg(y3p.reshape(N, H), ei4, z32)
    ps3 = psum3.reshape(NC, NP4, LW)[:, :N4]
    wh2p = jnp.zeros((H, H), jnp.float32).at[:, :3].set(Wh2.T)
    bh2p = jnp.zeros((H,), jnp.float32).at[:3].set(bh2)
    outp = _head(ps3, pc1, zb3,
                 _bd4(Wh1.T), _tile4(bh1), _bd4(wh2p), _tile4(bh2p))
    return outp.reshape(N, H)[:, :3]


# speedup vs baseline: 23.8682x; 1.0527x over previous
import functools

import numpy as np

import jax
import jax.numpy as jnp
from jax import lax
from jax.experimental import pallas as pl
from jax.experimental.pallas import tpu as pltpu
from jax.experimental.pallas import tpu_sc as plsc

N = 10000
E = 320000
D = 128
H = 32
P = 4
N4 = N // P
LW = P * H

NC = 2
NS = 16
NW = NC * NS
CHUNK = 128
NCH = 80
EPW = NCH * CHUNK
EPWR = E // NW
NBUF = 5
NPAD = 10240
NP4 = NPAD // P
ZR = NPAD // NS
SEG = N // NS
CW = 32
BR = 2000
B4 = BR // P


def _sc_agg_counts_body(y, ei4, z32, ones_h, out_sum, out_cnt,
                        sidx, didx, rows, ysp, acc, s0, s1, s2, s3, s4,
                        t0, t1, t2, t3, t4, ones_v, cacc, cs):
    cid = lax.axis_index("c")
    sid = lax.axis_index("s")
    wid = sid * NC + cid
    pltpu.sync_copy(z32, acc.at[pl.ds(sid * ZR, ZR)])
    pltpu.sync_copy(z32, cacc.at[pl.ds(sid * ZR, ZR)])
    pltpu.sync_copy(y.at[pl.ds(sid * SEG, SEG)], ysp.at[pl.ds(sid * SEG, SEG)])
    pltpu.sync_copy(ei4.at[0, wid], sidx)
    pltpu.sync_copy(ei4.at[1, wid], didx)
    pltpu.sync_copy(ones_h, ones_v)
    plsc.subcore_barrier()
    gsem = [s0, s1, s2, s3, s4]
    tsem = [t0, t1, t2, t3, t4]
    for b in range(NBUF):
        pltpu.async_copy(ysp.at[sidx.at[b]], rows.at[b], gsem[b])

    def group(g, carry):
        for b in range(NBUF):
            j = g * NBUF + b
            bp = (b - 1) % NBUF
            pltpu.make_async_copy(y.at[pl.ds(0, CHUNK)], rows.at[b], gsem[b]).wait()
            pltpu.async_copy(rows.at[b], acc.at[didx.at[j]], tsem[b], add=True)
            pltpu.async_copy(ones_v, cacc.at[didx.at[j]], cs, add=True)

            @pl.when(g >= 1)
            def _():
                pltpu.make_async_copy(ones_v, cacc.at[didx.at[0]], cs).wait()

            cond = (g >= 1) if b == 0 else (g <= NCH // NBUF - 2)

            @pl.when(cond)
            def _():
                jp = j - 1
                pltpu.make_async_copy(rows.at[bp], acc.at[didx.at[jp]],
                                      tsem[bp]).wait()
                pltpu.async_copy(ysp.at[sidx.at[jp + NBUF]], rows.at[bp], gsem[bp])
        return carry

    lax.fori_loop(0, NCH // NBUF, group, 0)
    for b in range(NBUF):
        pltpu.make_async_copy(rows.at[b], acc.at[didx.at[NCH - NBUF + b]],
                              tsem[b]).wait()
        pltpu.make_async_copy(ones_v, cacc.at[didx.at[0]], cs).wait()
    plsc.subcore_barrier()
    pltpu.sync_copy(acc.at[pl.ds(sid * ZR, ZR)], out_sum.at[cid, pl.ds(sid * ZR, ZR)])
    pltpu.sync_copy(cacc.at[pl.ds(sid * ZR, ZR)], out_cnt.at[cid, pl.ds(sid * ZR, ZR)])


def _sc_agg_body(y, ei4, z32, out_sum,
                 sidx, didx, rows, ysp, acc, s0, s1, s2, s3, s4,
                 t0, t1, t2, t3, t4):
    cid = lax.axis_index("c")
    sid = lax.axis_index("s")
    wid = sid * NC + cid
    pltpu.sync_copy(z32, acc.at[pl.ds(sid * ZR, ZR)])
    pltpu.sync_copy(y.at[pl.ds(sid * SEG, SEG)], ysp.at[pl.ds(sid * SEG, SEG)])
    pltpu.sync_copy(ei4.at[0, wid], sidx)
    pltpu.sync_copy(ei4.at[1, wid], didx)
    plsc.subcore_barrier()
    gsem = [s0, s1, s2, s3, s4]
    tsem = [t0, t1, t2, t3, t4]
    for b in range(NBUF):
        pltpu.async_copy(ysp.at[sidx.at[b]], rows.at[b], gsem[b])

    def group(g, carry):
        for b in range(NBUF):
            j = g * NBUF + b
            bp = (b - 1) % NBUF
            pltpu.make_async_copy(y.at[pl.ds(0, CHUNK)], rows.at[b], gsem[b]).wait()
            pltpu.async_copy(rows.at[b], acc.at[didx.at[j]], tsem[b], add=True)

            cond = (g >= 1) if b == 0 else (g <= NCH // NBUF - 2)

            @pl.when(cond)
            def _():
                jp = j - 1
                pltpu.make_async_copy(rows.at[bp], acc.at[didx.at[jp]],
                                      tsem[bp]).wait()
                pltpu.async_copy(ysp.at[sidx.at[jp + NBUF]], rows.at[bp], gsem[bp])
        return carry

    lax.fori_loop(0, NCH // NBUF, group, 0)
    for b in range(NBUF):
        pltpu.make_async_copy(rows.at[b], acc.at[didx.at[NCH - NBUF + b]],
                              tsem[b]).wait()
    plsc.subcore_barrier()
    pltpu.sync_copy(acc.at[pl.ds(sid * ZR, ZR)], out_sum.at[cid, pl.ds(sid * ZR, ZR)])


@functools.cache
def _sc_kernels():
    mesh = plsc.VectorSubcoreMesh(core_axis_name="c", subcore_axis_name="s",
                                  num_cores=NC, num_subcores=NS)
    common_scratch = [
        pltpu.VMEM((NCH, CHUNK), jnp.int32),
        pltpu.VMEM((NCH, CHUNK), jnp.int32),
        pltpu.VMEM((NBUF, CHUNK, H), jnp.float32),
        pltpu.VMEM_SHARED((N, H), jnp.float32),
        pltpu.VMEM_SHARED((NPAD, H), jnp.float32),
    ] + [pltpu.SemaphoreType.DMA] * (2 * NBUF)
    agg_counts = pl.kernel(
        _sc_agg_counts_body,
        out_type=[jax.ShapeDtypeStruct((NC, NPAD, H), jnp.float32),
                  jax.ShapeDtypeStruct((NC, NPAD, CW), jnp.float32)],
        mesh=mesh,
        compiler_params=pltpu.CompilerParams(use_tc_tiling_on_sc=False),
        scratch_types=common_scratch + [
            pltpu.VMEM((CHUNK, CW), jnp.float32),
            pltpu.VMEM_SHARED((NPAD, CW), jnp.float32),
            pltpu.SemaphoreType.DMA,
        ],
    )
    agg = pl.kernel(
        _sc_agg_body,
        out_type=[jax.ShapeDtypeStruct((NC, NPAD, H), jnp.float32)],
        mesh=mesh,
        compiler_params=pltpu.CompilerParams(use_tc_tiling_on_sc=False),
        scratch_types=common_scratch,
    )
    return agg_counts, agg


def _proj_body(x, wlT, wrT, bl, y_ref, z_ref):
    xb = x[...]
    y_ref[...] = jnp.dot(xb, wlT[...], preferred_element_type=jnp.float32)
    z_ref[...] = jnp.dot(xb, wrT[...], preferred_element_type=jnp.float32) + bl[...]


_proj = pl.pallas_call(
    _proj_body,
    out_shape=[jax.ShapeDtypeStruct((N, H), jnp.float32)] * 2,
)


def _comb_body(ps, pc, z, wlbd, wrbd, blp, y_ref, z_ref):
    inv = 1.0 / jnp.maximum(pc[0] + pc[1], 1.0)
    h = jnp.maximum((ps[0] + ps[1]) * inv + z[...], 0.0)
    y_ref[...] = jnp.dot(h, wlbd[...], preferred_element_type=jnp.float32)
    z_ref[...] = jnp.dot(h, wrbd[...], preferred_element_type=jnp.float32) + blp[...]


_comb = pl.pallas_call(
    _comb_body,
    out_shape=[jax.ShapeDtypeStruct((N4, LW), jnp.float32)] * 2,
)


def _head_body(ps, pc, z, wh1bd, bh1p, wh2bd, bh2p, o_ref):
    inv = 1.0 / jnp.maximum(pc[0] + pc[1], 1.0)
    h = jnp.maximum((ps[0] + ps[1]) * inv + z[...], 0.0)
    t = jnp.maximum(jnp.dot(h, wh1bd[...], preferred_element_type=jnp.float32)
                    + bh1p[...], 0.0)
    o_ref[...] = jnp.dot(t, wh2bd[...], preferred_element_type=jnp.float32) + bh2p[...]


_head = pl.pallas_call(
    _head_body,
    out_shape=jax.ShapeDtypeStruct((N4, LW), jnp.float32),
)


_EPAD = np.stack([
    np.zeros((NW, EPW - EPWR), np.int32),
    np.broadcast_to(N + np.arange(EPW - EPWR, dtype=np.int32), (NW, EPW - EPWR)),
])


def _bd4(w):
    return jax.scipy.linalg.block_diag(w, w, w, w)


def _tile4(b):
    return jnp.tile(b, P).reshape(1, LW)


def kernel(x, edge_index, Wl1, bl1, Wr1, Wl2, bl2, Wr2, Wl3, bl3, Wr3,
           Wh1, bh1, Wh2, bh2):
    _sc_agg_counts, _sc_agg = _sc_kernels()
    ei = edge_index.astype(jnp.int32).reshape(2, NW, EPWR)
    ei4 = jnp.concatenate([ei, _EPAD], axis=2).reshape(2, NW, NCH, CHUNK)
    z32 = np.zeros((ZR, H), np.float32)
    ones_h = np.ones((CHUNK, CW), np.float32)

    y1, zb1 = _proj(x, Wl1.T, Wr1.T, bl1.reshape(1, H))
    psum1, pcnt1 = _sc_agg_counts(y1, ei4, z32, ones_h)
    ps1 = psum1.reshape(NC, NP4, LW)[:, :N4]
    pc1 = pcnt1.reshape(NC, NP4, LW)[:, :N4]
    y2p, zb2 = _comb(ps1, pc1, zb1.reshape(N4, LW),
                     _bd4(Wl2.T), _bd4(Wr2.T), _tile4(bl2))
    (psum2,) = _sc_agg(y2p.reshape(N, H), ei4, z32)
    ps2 = psum2.reshape(NC, NP4, LW)[:, :N4]
    y3p, zb3 = _comb(ps2, pc1, zb2, _bd4(Wl3.T), _bd4(Wr3.T), _tile4(bl3))
    (psum3,) = _sc_agg(y3p.reshape(N, H), ei4, z32)
    ps3 = psum3.reshape(NC, NP4, LW)[:, :N4]
    wh2p = jnp.zeros((H, H), jnp.float32).at[:, :3].set(Wh2.T)
    bh2p = jnp.zeros((H,), jnp.float32).at[:3].set(bh2)
    outp = _head(ps3, pc1, zb3,
                 _bd4(Wh1.T), _tile4(bh1), _bd4(wh2p), _tile4(bh2p))
    return outp.reshape(N, H)[:, :3]
